# baseline (speedup 1.0000x reference)
import jax
import jax.numpy as jnp
from jax import lax
from jax.experimental import pallas as pl
from jax.experimental.pallas import tpu as pltpu

_A_ROWS = 4096
_B_ROWS = 2048
_LANES = 1024
_BR = 512
_NA = _A_ROWS // _BR
_NB = _B_ROWS // _BR
_WIN = _NB + 1


def _a_index_map(bb, k, i_ref):
    k0 = i_ref[0] // _BR
    interior = (k > k0) & (k < k0 + _NB)
    return bb, jnp.where(interior, k0, k), 0


def _j_of(k, i_ref):
    s = k * _BR - i_ref[0]
    return lax.div(s - jnp.where(s < 0, _BR - 1, 0), _BR)


def _b_even_index_map(bb, k, i_ref):
    j = _j_of(k, i_ref)
    e = j + (j & 1)
    return bb, jnp.clip(e, 0, _NB - 1), 0


def _b_odd_index_map(bb, k, i_ref):
    j = _j_of(k, i_ref)
    o = j + 1 - (j & 1)
    return bb, jnp.clip(o, 0, _NB - 1), 0


def _out_index_map(bb, k, i_ref):
    return bb, k, 0


def _body(i_ref, a_ref, be_ref, bo_ref, out_ref, sc_ref):
    k = pl.program_id(1)
    ii = i_ref[0]
    s = k * _BR - ii
    j = lax.div(s - jnp.where(s < 0, _BR - 1, 0), _BR)
    off = s - j * _BR
    q8 = (off // 8) * 8
    p = off - q8
    blk_start = k * _BR
    touches = (blk_start + _BR > ii) & (blk_start < ii + _B_ROWS)
    fully = (blk_start >= ii) & (blk_start + _BR <= ii + _B_ROWS)
    bdry = touches & jnp.logical_not(fully)
    j_even = (j & 1) == 0

    @pl.when(touches)
    def _():
        e_base = pl.multiple_of(jnp.where(j_even, 0, _BR), 8)
        o_base = pl.multiple_of(jnp.where(j_even, _BR, 0), 8)
        sc_ref[pl.ds(e_base, _BR), :] = be_ref[0]
        sc_ref[pl.ds(o_base, _BR), :] = bo_ref[0]

    q8a = pl.multiple_of(q8, 8)

    def sb_of(c):
        al = sc_ref[pl.ds(q8a, _BR + 8), :]
        return al[c:c + _BR]

    for c in range(8):
        @pl.when(fully & (p == c))
        def _(c=c):
            out_ref[0] = sb_of(c)

        @pl.when(bdry & (p == c))
        def _(c=c):
            riota = lax.broadcasted_iota(jnp.int32, (_BR, _LANES), 0)
            rows = blk_start + riota
            in_b = (rows >= ii) & (rows < ii + _B_ROWS)
            out_ref[0] = jnp.where(in_b, sb_of(c), a_ref[0])

    @pl.when(jnp.logical_not(touches))
    def _():
        out_ref[0] = a_ref[0]


def kernel(a, b, i):
    grid_spec = pltpu.PrefetchScalarGridSpec(
        num_scalar_prefetch=1,
        grid=(a.shape[0], _NA),
        in_specs=[
            pl.BlockSpec((1, _BR, _LANES), _a_index_map),
            pl.BlockSpec((1, _BR, _LANES), _b_even_index_map),
            pl.BlockSpec((1, _BR, _LANES), _b_odd_index_map),
        ],
        out_specs=pl.BlockSpec((1, _BR, _LANES), _out_index_map),
        scratch_shapes=[pltpu.VMEM((2 * _BR, _LANES), jnp.float32)],
    )
    return pl.pallas_call(
        _body,
        grid_spec=grid_spec,
        out_shape=jax.ShapeDtypeStruct(a.shape, a.dtype),
        compiler_params=pltpu.CompilerParams(
            dimension_semantics=("parallel", "arbitrary"),
        ),
    )(i, a, b, b)
